# baseline (speedup 1.0000x reference)
import functools

import jax
import jax.numpy as jnp
from jax import lax
from jax.experimental import pallas as pl
from jax.experimental.pallas import tpu as pltpu
from jax.experimental.pallas import tpu_sc as plsc

_NUM_CORES = 2
_NUM_SUBCORES = 16
_NW = _NUM_CORES * _NUM_SUBCORES


@functools.lru_cache(maxsize=None)
def _build(B0, S, D):
    rows_per_w = B0 // _NW
    steps = rows_per_w
    mesh = plsc.VectorSubcoreMesh(core_axis_name="c", subcore_axis_name="s")

    @functools.partial(
        pl.kernel,
        mesh=mesh,
        out_type=jax.ShapeDtypeStruct((B0, S, D), jnp.float32),
        compiler_params=pltpu.CompilerParams(
            use_tc_tiling_on_sc=False, skip_device_barrier=True),
        scratch_types=[
            pltpu.VMEM((rows_per_w, S), jnp.int32),
            pltpu.VMEM((2, S, D), jnp.float32),
            pltpu.SemaphoreType.DMA,
            pltpu.SemaphoreType.DMA,
            pltpu.SemaphoreType.DMA,
            pltpu.SemaphoreType.DMA,
        ],
    )
    def k(x_hbm, table_hbm, out_hbm, idx_v, rows_v, g0, g1, o0, o1):
        gsem = (g0, g1)
        osem = (o0, o1)
        wid = lax.axis_index("s") * _NUM_CORES + lax.axis_index("c")
        base = wid * rows_per_w
        pltpu.sync_copy(x_hbm.at[pl.ds(base, rows_per_w)], idx_v)

        def gather_start(i, b):
            pltpu.async_copy(table_hbm.at[idx_v.at[i]], rows_v.at[b], gsem[b])

        def gather_wait(i, b):
            pltpu.make_async_copy(
                table_hbm.at[idx_v.at[i]], rows_v.at[b], gsem[b]).wait()

        def store_start(i, b):
            pltpu.async_copy(rows_v.at[b], out_hbm.at[base + i], osem[b])

        def store_wait(i, b):
            pltpu.make_async_copy(
                rows_v.at[b], out_hbm.at[base + i], osem[b]).wait()

        gather_start(0, 0)

        def body(j, carry):
            for b in range(2):
                i = j * 2 + b
                ob = 1 - b
                gather_wait(i, b)

                @pl.when(i + 1 < steps)
                def _():
                    @pl.when(i >= 1)
                    def _():
                        store_wait(i - 1, ob)

                    gather_start(i + 1, ob)

                store_start(i, b)
            return carry

        lax.fori_loop(0, steps // 2, body, 0)
        store_wait(steps - 2, 0)
        store_wait(steps - 1, 1)

    return k


def kernel(x, table):
    B0, S = x.shape
    V, D = table.shape
    return _build(B0, S, D)(x, table)


# speedup vs baseline: 1.2477x; 1.2477x over previous
import functools

import jax
import jax.numpy as jnp
from jax import lax
from jax.experimental import pallas as pl
from jax.experimental.pallas import tpu as pltpu
from jax.experimental.pallas import tpu_sc as plsc

_NUM_CORES = 2
_NUM_SUBCORES = 16
_NW = _NUM_CORES * _NUM_SUBCORES


@functools.lru_cache(maxsize=None)
def _build(B0, S, D, DP):
    rows_per_w = B0 // _NW
    steps = rows_per_w
    mesh = plsc.VectorSubcoreMesh(core_axis_name="c", subcore_axis_name="s")

    @functools.partial(
        pl.kernel,
        mesh=mesh,
        out_type=jax.ShapeDtypeStruct((B0, S, DP), jnp.float32),
        compiler_params=pltpu.CompilerParams(use_tc_tiling_on_sc=True),
        scratch_types=[
            pltpu.VMEM((S,), jnp.int32),
            pltpu.VMEM((S,), jnp.int32),
            pltpu.VMEM((2, S, DP), jnp.float32),
            pltpu.SemaphoreType.DMA,
            pltpu.SemaphoreType.DMA,
            pltpu.SemaphoreType.DMA,
            pltpu.SemaphoreType.DMA,
            pltpu.SemaphoreType.DMA,
            pltpu.SemaphoreType.DMA,
        ],
    )
    def k(x_hbm, table_hbm, out_hbm, ia, ib, rows_v, i0, i1, g0, g1, o0, o1):
        idxv = (ia, ib)
        isem = (i0, i1)
        gsem = (g0, g1)
        osem = (o0, o1)
        wid = lax.axis_index("s") * _NUM_CORES + lax.axis_index("c")
        base = wid * rows_per_w

        def idx_start(i, b):
            pltpu.async_copy(x_hbm.at[base + i], idxv[b], isem[b])

        def idx_wait(i, b):
            pltpu.make_async_copy(x_hbm.at[base + i], idxv[b], isem[b]).wait()

        def gather_start(i, b):
            pltpu.async_copy(table_hbm.at[idxv[b]], rows_v.at[b], gsem[b])

        def gather_wait(i, b):
            pltpu.make_async_copy(
                table_hbm.at[idxv[b]], rows_v.at[b], gsem[b]).wait()

        def store_start(i, b):
            pltpu.async_copy(rows_v.at[b], out_hbm.at[base + i], osem[b])

        def store_wait(i, b):
            pltpu.make_async_copy(
                rows_v.at[b], out_hbm.at[base + i], osem[b]).wait()

        idx_start(0, 0)
        idx_wait(0, 0)
        gather_start(0, 0)
        idx_start(1, 1)

        def body(j, carry):
            for b in range(2):
                i = j * 2 + b
                ob = 1 - b
                gather_wait(i, b)

                @pl.when(i + 2 < steps)
                def _():
                    idx_start(i + 2, b)

                @pl.when(i + 1 < steps)
                def _():
                    @pl.when(i >= 1)
                    def _():
                        store_wait(i - 1, ob)

                    idx_wait(i + 1, ob)
                    gather_start(i + 1, ob)

                store_start(i, b)
            return carry

        lax.fori_loop(0, steps // 2, body, 0)
        store_wait(steps - 2, 0)
        store_wait(steps - 1, 1)

    return k


def kernel(x, table):
    B0, S = x.shape
    V, D = table.shape
    DP = 128
    table_p = jnp.pad(table, ((0, 0), (0, DP - D)))
    out = _build(B0, S, D, DP)(x, table_p)
    return out[:, :, :D]


# speedup vs baseline: 1.4057x; 1.1266x over previous
import functools

import jax
import jax.numpy as jnp
from jax import lax
from jax.experimental import pallas as pl
from jax.experimental.pallas import tpu as pltpu
from jax.experimental.pallas import tpu_sc as plsc

_NUM_CORES = 2
_NUM_SUBCORES = 16
_NW = _NUM_CORES * _NUM_SUBCORES


@functools.lru_cache(maxsize=None)
def _build(B0, S, D, DP):
    rows_per_w = B0 // _NW
    steps = rows_per_w
    mesh = plsc.VectorSubcoreMesh(core_axis_name="c", subcore_axis_name="s")

    @functools.partial(
        pl.kernel,
        mesh=mesh,
        out_type=jax.ShapeDtypeStruct((B0, S, DP), jnp.float32),
        compiler_params=pltpu.CompilerParams(use_tc_tiling_on_sc=False),
        scratch_types=[
            pltpu.VMEM((S,), jnp.int32),
            pltpu.VMEM((S,), jnp.int32),
            pltpu.VMEM((2, S, D), jnp.float32),
            pltpu.SemaphoreType.DMA,
            pltpu.SemaphoreType.DMA,
            pltpu.SemaphoreType.DMA,
            pltpu.SemaphoreType.DMA,
            pltpu.SemaphoreType.DMA,
            pltpu.SemaphoreType.DMA,
        ],
    )
    def k(x_hbm, table_hbm, out_hbm, ia, ib, rows_v, i0, i1, g0, g1, o0, o1):
        idxv = (ia, ib)
        isem = (i0, i1)
        gsem = (g0, g1)
        osem = (o0, o1)
        wid = lax.axis_index("s") * _NUM_CORES + lax.axis_index("c")
        base = wid * rows_per_w

        def idx_start(i, b):
            pltpu.async_copy(x_hbm.at[base + i], idxv[b], isem[b])

        def idx_wait(i, b):
            pltpu.make_async_copy(x_hbm.at[base + i], idxv[b], isem[b]).wait()

        def gather_start(i, b):
            pltpu.async_copy(table_hbm.at[idxv[b]], rows_v.at[b], gsem[b])

        def gather_wait(i, b):
            pltpu.make_async_copy(
                table_hbm.at[idxv[b]], rows_v.at[b], gsem[b]).wait()

        def store_start(i, b):
            pltpu.async_copy(
                rows_v.at[b], out_hbm.at[base + i, :, pl.ds(0, D)], osem[b])

        def store_wait(i, b):
            pltpu.make_async_copy(
                rows_v.at[b], out_hbm.at[base + i, :, pl.ds(0, D)],
                osem[b]).wait()

        idx_start(0, 0)
        idx_wait(0, 0)
        gather_start(0, 0)
        idx_start(1, 1)

        def body(j, carry):
            for b in range(2):
                i = j * 2 + b
                ob = 1 - b
                gather_wait(i, b)

                @pl.when(i + 2 < steps)
                def _():
                    idx_start(i + 2, b)

                @pl.when(i + 1 < steps)
                def _():
                    @pl.when(i >= 1)
                    def _():
                        store_wait(i - 1, ob)

                    idx_wait(i + 1, ob)
                    gather_start(i + 1, ob)

                store_start(i, b)
            return carry

        lax.fori_loop(0, steps // 2, body, 0)
        store_wait(steps - 2, 0)
        store_wait(steps - 1, 1)

    return k


def kernel(x, table):
    B0, S = x.shape
    V, D = table.shape
    DP = 128
    table_p = jnp.pad(table, ((0, 0), (0, DP - D)))
    table_v = table_p.reshape(V * (DP // D), D)
    x2 = x * (DP // D)
    out = _build(B0, S, D, DP)(x2, table_v)
    return out[:, :, :D]
